# baseline (speedup 1.0000x reference)
import jax
import jax.numpy as jnp
from jax import lax
from jax.experimental import pallas as pl

NUM_EXPERTS = 64
TOP_K = 8
BLOCK_M = 512


def _topk_store(p, wout_ref, iout_ref):
    bm = p.shape[0]
    iota = lax.broadcasted_iota(jnp.int32, (bm, NUM_EXPERTS), 1)
    cur = p
    ws, ids = [], []
    for _ in range(TOP_K):
        mx = jnp.max(cur, axis=1, keepdims=True)
        amx = jnp.min(jnp.where(cur == mx, iota, NUM_EXPERTS), axis=1, keepdims=True)
        ws.append(mx)
        ids.append(amx)
        cur = jnp.where(iota == amx, -jnp.inf, cur)
    wout_ref[...] = jnp.concatenate(ws, axis=1)
    iout_ref[...] = jnp.concatenate(ids, axis=1)


def _router_block(x1_ref, x2_ref, w_ref, wout1_ref, iout1_ref, wout2_ref, iout2_ref):
    w = w_ref[...]
    for x_ref, wo, io in ((x1_ref, wout1_ref, iout1_ref),
                          (x2_ref, wout2_ref, iout2_ref)):
        logits = lax.dot_general(
            x_ref[...], w,
            dimension_numbers=(((1,), (0,)), ((), ())),
            preferred_element_type=jnp.float32,
        )
        m = jnp.max(logits, axis=1, keepdims=True)
        e = jnp.exp(logits - m)
        p = e / jnp.sum(e, axis=1, keepdims=True)
        _topk_store(p, wo, io)


def kernel(x, W):
    h = x.reshape(-1, x.shape[-1])
    M, K = h.shape
    E = W.shape[0]
    Wt = jnp.swapaxes(W, 0, 1)
    half = M // 2
    h1, h2 = h[:half], h[half:]
    bm = BLOCK_M
    grid = (half // bm,)
    w1, i1, w2, i2 = pl.pallas_call(
        _router_block,
        grid=grid,
        in_specs=[
            pl.BlockSpec((bm, K), lambda i: (i, 0)),
            pl.BlockSpec((bm, K), lambda i: (i, 0)),
            pl.BlockSpec((K, E), lambda i: (0, 0)),
        ],
        out_specs=[
            pl.BlockSpec((bm, TOP_K), lambda i: (i, 0)),
            pl.BlockSpec((bm, TOP_K), lambda i: (i, 0)),
            pl.BlockSpec((bm, TOP_K), lambda i: (i, 0)),
            pl.BlockSpec((bm, TOP_K), lambda i: (i, 0)),
        ],
        out_shape=[
            jax.ShapeDtypeStruct((half, TOP_K), jnp.float32),
            jax.ShapeDtypeStruct((half, TOP_K), jnp.int32),
            jax.ShapeDtypeStruct((half, TOP_K), jnp.float32),
            jax.ShapeDtypeStruct((half, TOP_K), jnp.int32),
        ],
    )(h1, h2, Wt)
    return (jnp.concatenate([w1, w2], axis=0), jnp.concatenate([i1, i2], axis=0))


# speedup vs baseline: 1.7386x; 1.7386x over previous
import jax
import jax.numpy as jnp
from jax import lax
from jax.experimental import pallas as pl

NUM_EXPERTS = 64
TOP_K = 8
BLOCK_M = 512


def _topk_store(p, wout_ref, iout_ref):
    bm = p.shape[0]
    iota = lax.broadcasted_iota(jnp.int32, (bm, NUM_EXPERTS), 1)
    cur = p
    ws, ids = [], []
    for _ in range(TOP_K):
        mx = jnp.max(cur, axis=1, keepdims=True)
        amx = jnp.min(jnp.where(cur == mx, iota, NUM_EXPERTS), axis=1, keepdims=True)
        ws.append(mx)
        ids.append(amx)
        cur = jnp.where(iota == amx, -jnp.inf, cur)
    wout_ref[...] = jnp.concatenate(ws, axis=1)
    iout_ref[...] = jnp.concatenate(ids, axis=1)


def _router_block(x1_ref, x2_ref, w_ref, wout1_ref, iout1_ref, wout2_ref, iout2_ref):
    w = w_ref[...]
    for x_ref, wo, io in ((x1_ref, wout1_ref, iout1_ref),
                          (x2_ref, wout2_ref, iout2_ref)):
        logits = lax.dot_general(
            x_ref[...], w,
            dimension_numbers=(((1,), (0,)), ((), ())),
            preferred_element_type=jnp.float32,
        )
        m = jnp.max(logits, axis=1, keepdims=True)
        e = jnp.exp(logits - m)
        p = e / jnp.sum(e, axis=1, keepdims=True)
        _topk_store(p, wo, io)


def kernel(x, W):
    h = x.reshape(-1, x.shape[-1])
    M, K = h.shape
    E = W.shape[0]
    Wt = jnp.swapaxes(W, 0, 1)
    half = M // 2
    bm = BLOCK_M
    grid = (half // bm,)
    nb1 = half // bm
    w1, i1, w2, i2 = pl.pallas_call(
        _router_block,
        grid=grid,
        in_specs=[
            pl.BlockSpec((bm, K), lambda i: (i, 0)),
            pl.BlockSpec((bm, K), lambda i: (i + nb1, 0)),
            pl.BlockSpec((K, E), lambda i: (0, 0)),
        ],
        out_specs=[
            pl.BlockSpec((bm, TOP_K), lambda i: (i, 0)),
            pl.BlockSpec((bm, TOP_K), lambda i: (i, 0)),
            pl.BlockSpec((bm, TOP_K), lambda i: (i, 0)),
            pl.BlockSpec((bm, TOP_K), lambda i: (i, 0)),
        ],
        out_shape=[
            jax.ShapeDtypeStruct((half, TOP_K), jnp.float32),
            jax.ShapeDtypeStruct((half, TOP_K), jnp.int32),
            jax.ShapeDtypeStruct((half, TOP_K), jnp.float32),
            jax.ShapeDtypeStruct((half, TOP_K), jnp.int32),
        ],
    )(h, h, Wt)
    return (jnp.concatenate([w1, w2], axis=0), jnp.concatenate([i1, i2], axis=0))


# speedup vs baseline: 1.7507x; 1.0069x over previous
import jax
import jax.numpy as jnp
from jax import lax
from jax.experimental import pallas as pl

NUM_EXPERTS = 64
TOP_K = 8
BLOCK_M = 1024


def _router_block(x_ref, w_ref, wout_ref, iout_ref):
    bm = x_ref.shape[0]
    logits = lax.dot_general(
        x_ref[...], w_ref[...],
        dimension_numbers=(((1,), (0,)), ((), ())),
        preferred_element_type=jnp.float32,
        precision=lax.Precision.DEFAULT,
    )
    m = jnp.max(logits, axis=1, keepdims=True)
    e = jnp.exp(logits - m)
    p = e / jnp.sum(e, axis=1, keepdims=True)

    iota = lax.broadcasted_iota(jnp.int32, (bm, NUM_EXPERTS), 1)
    cur = p
    ws, ids = [], []
    for _ in range(TOP_K):
        mx = jnp.max(cur, axis=1, keepdims=True)
        amx = jnp.min(jnp.where(cur == mx, iota, NUM_EXPERTS), axis=1, keepdims=True)
        ws.append(mx)
        ids.append(amx)
        cur = jnp.where(iota == amx, -jnp.inf, cur)
    wout_ref[...] = jnp.concatenate(ws, axis=1)
    iout_ref[...] = jnp.concatenate(ids, axis=1)


def kernel(x, W):
    h = x.reshape(-1, x.shape[-1])
    M, K = h.shape
    E = W.shape[0]
    Wt = jnp.swapaxes(W, 0, 1)
    bm = BLOCK_M if M % BLOCK_M == 0 else 256
    grid = (M // bm,)
    wout, iout = pl.pallas_call(
        _router_block,
        grid=grid,
        in_specs=[
            pl.BlockSpec((bm, K), lambda i: (i, 0)),
            pl.BlockSpec((K, E), lambda i: (0, 0)),
        ],
        out_specs=[
            pl.BlockSpec((bm, TOP_K), lambda i: (i, 0)),
            pl.BlockSpec((bm, TOP_K), lambda i: (i, 0)),
        ],
        out_shape=[
            jax.ShapeDtypeStruct((M, TOP_K), jnp.float32),
            jax.ShapeDtypeStruct((M, TOP_K), jnp.int32),
        ],
    )(h, Wt)
    return (wout, iout)
